# Initial kernel scaffold; baseline (speedup 1.0000x reference)
import jax
import jax.numpy as jnp
from jax.experimental import pallas as pl


def kernel(m_st, rbf, cbf, idx_s, idx_swap, basis_idx1, W_m_rbf, W_rbf, W_m_cbf, W_cbf, W_dir, W_st, W_ts):
    raise NotImplementedError("write your pallas kernel here")


# speedup vs baseline: 2.6127x; 2.6127x over previous
import functools
import math

import jax
import jax.numpy as jnp
from jax import lax
from jax.experimental import pallas as pl
from jax.experimental.pallas import tpu as pltpu
from jax.experimental.pallas import tpu_sc as plsc

_NW = 32
_CH = 80
_INV_SQRT2 = 1.0 / math.sqrt(2.0)


def _sc_gather(table, idx):
    B = idx.shape[0]
    D = table.shape[1]
    nch = B // _CH
    cpw = nch // _NW
    assert cpw * _NW * _CH == B
    mesh = plsc.VectorSubcoreMesh(core_axis_name="c", subcore_axis_name="s")

    @functools.partial(
        pl.kernel,
        mesh=mesh,
        out_type=jax.ShapeDtypeStruct((B, D), table.dtype),
        scratch_types=[
            pltpu.VMEM((_CH,), jnp.int32),
            pltpu.VMEM((_CH, D), table.dtype),
            pltpu.SemaphoreType.DMA,
        ],
    )
    def k(table_hbm, idx_hbm, out_hbm, idx_v, rows_v, sem):
        w = lax.axis_index("s") * 2 + lax.axis_index("c")

        def body(i, carry):
            base = (w * cpw + i) * _CH
            pltpu.sync_copy(idx_hbm.at[pl.ds(base, _CH)], idx_v)
            pltpu.async_copy(table_hbm.at[idx_v], rows_v, sem).wait()
            pltpu.sync_copy(rows_v, out_hbm.at[pl.ds(base, _CH)])
            return carry

        lax.fori_loop(0, cpw, body, 0)

    return k(table, idx)


def _tc1_body(m_st_ref, rbf_ref, w1_ref, w2_ref, w3_ref, out_ref):
    m = jax.nn.silu(jnp.dot(m_st_ref[...], w1_ref[...],
                            preferred_element_type=jnp.float32))
    m = m * jnp.dot(rbf_ref[...], w2_ref[...],
                    preferred_element_type=jnp.float32)
    out_ref[...] = jax.nn.silu(jnp.dot(m, w3_ref[...],
                                       preferred_element_type=jnp.float32))


def _tc2_body(gg_ref, cbf_ref, wcbf_ref, wdir_ref, wts_ref, x_ref, xts_ref):
    gg = gg_ref[...]
    cbf = cbf_ref[...]
    wcbf = wcbf_ref[...]
    acc = None
    for nb in range(4):
        p = jnp.dot(cbf[:, nb * 16:(nb + 1) * 16], wcbf,
                    preferred_element_type=jnp.float32)
        t = gg[:, nb * 64:(nb + 1) * 64] * p
        acc = t if acc is None else acc + t
    x = acc * 0.5
    x = jax.nn.silu(jnp.dot(x, wdir_ref[...],
                            preferred_element_type=jnp.float32))
    x_ref[...] = x
    xts_ref[...] = jax.nn.silu(jnp.dot(x, wts_ref[...],
                                       preferred_element_type=jnp.float32)) * _INV_SQRT2


def _tc3_body(x_ref, xg_ref, wst_ref, out_ref):
    xst = jax.nn.silu(jnp.dot(x_ref[...], wst_ref[...],
                              preferred_element_type=jnp.float32))
    out_ref[...] = xst * _INV_SQRT2 + xg_ref[...]


def _full(shape):
    return pl.BlockSpec(shape, lambda i: (0, 0))


def _rows(r, d):
    return pl.BlockSpec((r, d), lambda i: (i, 0))


def kernel(m_st, rbf, cbf, idx_s, idx_swap, basis_idx1, W_m_rbf, W_rbf,
           W_m_cbf, W_cbf, W_dir, W_st, W_ts):
    E, D_EDGE = m_st.shape
    N, NB = basis_idx1.shape
    D_TRI = W_m_cbf.shape[1]
    D_RBF = rbf.shape[1]
    D_CBF = cbf.shape[2]
    R = 512
    grid = (E // R,)

    w3p = jnp.pad(W_m_cbf, ((0, 0), (0, 128 - D_TRI)))
    m_nb = pl.pallas_call(
        _tc1_body,
        grid=grid,
        in_specs=[_rows(R, D_EDGE), _rows(R, D_RBF), _full((D_EDGE, D_EDGE)),
                  _full((D_RBF, D_EDGE)), _full((D_EDGE, 128))],
        out_specs=_rows(R, 128),
        out_shape=jax.ShapeDtypeStruct((E, 128), jnp.float32),
    )(m_st, rbf, W_m_rbf, W_rbf, w3p)

    bflat = basis_idx1.reshape(-1)
    npad = (-bflat.shape[0]) % (_CH * _NW)
    bpad = jnp.concatenate([bflat, jnp.zeros((npad,), jnp.int32)])
    Gp = _sc_gather(m_nb, bpad)
    Gr = Gp[:N * NB, :D_TRI].reshape(N, NB * D_TRI)

    Gg = _sc_gather(Gr, idx_s)

    cbf2d = cbf.reshape(E, NB * D_CBF)
    x, xts = pl.pallas_call(
        _tc2_body,
        grid=grid,
        in_specs=[_rows(R, NB * D_TRI), _rows(R, NB * D_CBF),
                  _full((D_CBF, D_TRI)), _full((D_TRI, D_EDGE)),
                  _full((D_EDGE, D_EDGE))],
        out_specs=[_rows(R, D_EDGE), _rows(R, D_EDGE)],
        out_shape=[jax.ShapeDtypeStruct((E, D_EDGE), jnp.float32),
                   jax.ShapeDtypeStruct((E, D_EDGE), jnp.float32)],
    )(Gg, cbf2d, W_cbf, W_dir, W_ts)

    xg = _sc_gather(xts, idx_swap)

    out = pl.pallas_call(
        _tc3_body,
        grid=grid,
        in_specs=[_rows(R, D_EDGE), _rows(R, D_EDGE), _full((D_EDGE, D_EDGE))],
        out_specs=_rows(R, D_EDGE),
        out_shape=jax.ShapeDtypeStruct((E, D_EDGE), jnp.float32),
    )(x, xg, W_st)
    return out


# speedup vs baseline: 3.0066x; 1.1508x over previous
import functools
import math

import jax
import jax.numpy as jnp
from jax import lax
from jax.experimental import pallas as pl
from jax.experimental.pallas import tpu as pltpu
from jax.experimental.pallas import tpu_sc as plsc

_NW = 32
_CH = 80
_INV_SQRT2 = 1.0 / math.sqrt(2.0)


def _sc_gather(table, idx):
    B = idx.shape[0]
    D = table.shape[1]
    nch = B // _CH
    cpw = nch // _NW
    assert cpw * _NW * _CH == B
    mesh = plsc.VectorSubcoreMesh(core_axis_name="c", subcore_axis_name="s")

    @functools.partial(
        pl.kernel,
        mesh=mesh,
        out_type=jax.ShapeDtypeStruct((B, D), table.dtype),
        scratch_types=[
            pltpu.VMEM((_CH,), jnp.int32),
            pltpu.VMEM((_CH, D), table.dtype),
            pltpu.SemaphoreType.DMA,
        ],
    )
    def k(table_hbm, idx_hbm, out_hbm, idx_v, rows_v, sem):
        w = lax.axis_index("s") * 2 + lax.axis_index("c")

        def body(i, carry):
            base = (w * cpw + i) * _CH
            pltpu.sync_copy(idx_hbm.at[pl.ds(base, _CH)], idx_v)
            pltpu.async_copy(table_hbm.at[idx_v], rows_v, sem).wait()
            pltpu.sync_copy(rows_v, out_hbm.at[pl.ds(base, _CH)])
            return carry

        lax.fori_loop(0, cpw, body, 0)

    return k(table, idx)


def _sc_gather_add(xst, xts, idx):
    E, D = xst.shape
    nch = E // _CH
    cpw = nch // _NW
    assert cpw * _NW * _CH == E and D % 16 == 0
    mesh = plsc.VectorSubcoreMesh(core_axis_name="c", subcore_axis_name="s")

    @functools.partial(
        pl.kernel,
        mesh=mesh,
        out_type=jax.ShapeDtypeStruct((E, D), jnp.float32),
        scratch_types=[
            pltpu.VMEM((_CH,), jnp.int32),
            pltpu.VMEM((_CH, D), jnp.float32),
            pltpu.VMEM((_CH, D), jnp.float32),
            pltpu.SemaphoreType.DMA,
        ],
    )
    def k(xst_hbm, xts_hbm, idx_hbm, out_hbm, idx_v, acc_v, rows_v, sem):
        w = lax.axis_index("s") * 2 + lax.axis_index("c")

        def body(i, carry):
            base = (w * cpw + i) * _CH
            pltpu.sync_copy(idx_hbm.at[pl.ds(base, _CH)], idx_v)
            gather = pltpu.async_copy(xts_hbm.at[idx_v], rows_v, sem)
            pltpu.sync_copy(xst_hbm.at[pl.ds(base, _CH)], acc_v)
            gather.wait()

            def add_row(j, c2):
                for kk in range(D // 16):
                    s = pl.ds(kk * 16, 16)
                    acc_v[j, s] = acc_v[j, s] + rows_v[j, s]
                return c2

            lax.fori_loop(0, _CH, add_row, 0)
            pltpu.sync_copy(acc_v, out_hbm.at[pl.ds(base, _CH)])
            return carry

        lax.fori_loop(0, cpw, body, 0)

    return k(xst, xts, idx)


def _tc1_body(m_st_ref, rbf_ref, w1_ref, w2_ref, w3_ref, out_ref):
    m = jax.nn.silu(jnp.dot(m_st_ref[...], w1_ref[...],
                            preferred_element_type=jnp.float32))
    m = m * jnp.dot(rbf_ref[...], w2_ref[...],
                    preferred_element_type=jnp.float32)
    out_ref[...] = jax.nn.silu(jnp.dot(m, w3_ref[...],
                                       preferred_element_type=jnp.float32))


def _tc2_body(gg_ref, cbf_ref, wcbf_ref, wdir_ref, wst_ref, wts_ref,
              xst_ref, xts_ref):
    gg = gg_ref[...].astype(jnp.float32)
    cbf = cbf_ref[...]
    wcbf = wcbf_ref[...]
    acc = None
    for nb in range(4):
        p = jnp.dot(cbf[:, nb * 16:(nb + 1) * 16], wcbf,
                    preferred_element_type=jnp.float32)
        t = gg[:, nb * 64:(nb + 1) * 64] * p
        acc = t if acc is None else acc + t
    x = acc * 0.5
    x = jax.nn.silu(jnp.dot(x, wdir_ref[...],
                            preferred_element_type=jnp.float32))
    xst_ref[...] = jax.nn.silu(jnp.dot(x, wst_ref[...],
                                       preferred_element_type=jnp.float32)) * _INV_SQRT2
    xts_ref[...] = jax.nn.silu(jnp.dot(x, wts_ref[...],
                                       preferred_element_type=jnp.float32)) * _INV_SQRT2


def _full(shape):
    return pl.BlockSpec(shape, lambda i: (0, 0))


def _rows(r, d):
    return pl.BlockSpec((r, d), lambda i: (i, 0))


def kernel(m_st, rbf, cbf, idx_s, idx_swap, basis_idx1, W_m_rbf, W_rbf,
           W_m_cbf, W_cbf, W_dir, W_st, W_ts):
    E, D_EDGE = m_st.shape
    N, NB = basis_idx1.shape
    D_TRI = W_m_cbf.shape[1]
    D_RBF = rbf.shape[1]
    D_CBF = cbf.shape[2]
    R = 512
    grid = (E // R,)

    w3p = jnp.pad(W_m_cbf, ((0, 0), (0, 128 - D_TRI)))
    m_nb = pl.pallas_call(
        _tc1_body,
        grid=grid,
        in_specs=[_rows(R, D_EDGE), _rows(R, D_RBF), _full((D_EDGE, D_EDGE)),
                  _full((D_RBF, D_EDGE)), _full((D_EDGE, 128))],
        out_specs=_rows(R, 128),
        out_shape=jax.ShapeDtypeStruct((E, 128), jnp.float32),
    )(m_st, rbf, W_m_rbf, W_rbf, w3p)

    bflat = basis_idx1.reshape(-1)
    npad = (-bflat.shape[0]) % (_CH * _NW)
    bpad = jnp.concatenate([bflat, jnp.zeros((npad,), jnp.int32)])
    Gp = _sc_gather(m_nb, bpad)
    Gr = Gp[:N * NB, :D_TRI].reshape(N, NB * D_TRI)

    Gg = _sc_gather(Gr, idx_s)

    cbf2d = cbf.reshape(E, NB * D_CBF)
    xst, xts = pl.pallas_call(
        _tc2_body,
        grid=grid,
        in_specs=[_rows(R, NB * D_TRI), _rows(R, NB * D_CBF),
                  _full((D_CBF, D_TRI)), _full((D_TRI, D_EDGE)),
                  _full((D_EDGE, D_EDGE)), _full((D_EDGE, D_EDGE))],
        out_specs=[_rows(R, D_EDGE), _rows(R, D_EDGE)],
        out_shape=[jax.ShapeDtypeStruct((E, D_EDGE), jnp.float32),
                   jax.ShapeDtypeStruct((E, D_EDGE), jnp.float32)],
    )(Gg, cbf2d, W_cbf, W_dir, W_st, W_ts)

    return _sc_gather_add(xst, xts, idx_swap)


# speedup vs baseline: 3.2196x; 1.0709x over previous
import functools
import math

import jax
import jax.numpy as jnp
from jax import lax
from jax.experimental import pallas as pl
from jax.experimental.pallas import tpu as pltpu
from jax.experimental.pallas import tpu_sc as plsc

_NW = 32
_CH = 80
_INV_SQRT2 = 1.0 / math.sqrt(2.0)


def _sc_gather(table, idx):
    B = idx.shape[0]
    D = table.shape[1]
    nch = B // _CH
    cpw = nch // _NW
    assert cpw * _NW * _CH == B
    mesh = plsc.VectorSubcoreMesh(core_axis_name="c", subcore_axis_name="s")

    @functools.partial(
        pl.kernel,
        mesh=mesh,
        out_type=jax.ShapeDtypeStruct((B, D), table.dtype),
        scratch_types=[
            pltpu.VMEM((_CH,), jnp.int32),
            pltpu.VMEM((_CH, D), table.dtype),
            pltpu.SemaphoreType.DMA,
        ],
    )
    def k(table_hbm, idx_hbm, out_hbm, idx_v, rows_v, sem):
        w = lax.axis_index("s") * 2 + lax.axis_index("c")

        def body(i, carry):
            base = (w * cpw + i) * _CH
            pltpu.sync_copy(idx_hbm.at[pl.ds(base, _CH)], idx_v)
            pltpu.async_copy(table_hbm.at[idx_v], rows_v, sem).wait()
            pltpu.sync_copy(rows_v, out_hbm.at[pl.ds(base, _CH)])
            return carry

        lax.fori_loop(0, cpw, body, 0)

    return k(table, idx)


def _sc_gather_add(xst, xts, idx):
    E, D = xst.shape
    nch = E // _CH
    cpw = nch // _NW
    assert cpw * _NW * _CH == E and D % 16 == 0
    mesh = plsc.VectorSubcoreMesh(core_axis_name="c", subcore_axis_name="s")

    @functools.partial(
        pl.kernel,
        mesh=mesh,
        out_type=jax.ShapeDtypeStruct((E, D), jnp.float32),
        scratch_types=[
            pltpu.VMEM((_CH,), jnp.int32),
            pltpu.VMEM((_CH, D), jnp.float32),
            pltpu.VMEM((_CH, D), jnp.float32),
            pltpu.SemaphoreType.DMA,
        ],
    )
    def k(xst_hbm, xts_hbm, idx_hbm, out_hbm, idx_v, acc_v, rows_v, sem):
        w = lax.axis_index("s") * 2 + lax.axis_index("c")

        def body(i, carry):
            base = (w * cpw + i) * _CH
            pltpu.sync_copy(idx_hbm.at[pl.ds(base, _CH)], idx_v)
            gather = pltpu.async_copy(xts_hbm.at[idx_v], rows_v, sem)
            pltpu.sync_copy(xst_hbm.at[pl.ds(base, _CH)], acc_v)
            gather.wait()

            def add_row(j, c2):
                for kk in range(D // 16):
                    s = pl.ds(kk * 16, 16)
                    acc_v[j, s] = acc_v[j, s] + rows_v[j, s]
                return c2

            lax.fori_loop(0, _CH, add_row, 0)
            pltpu.sync_copy(acc_v, out_hbm.at[pl.ds(base, _CH)])
            return carry

        lax.fori_loop(0, cpw, body, 0)

    return k(xst, xts, idx)


def _tc1_body(m_st_ref, rbf_ref, w1_ref, w2_ref, w3_ref, out_ref):
    m = jax.nn.silu(jnp.dot(m_st_ref[...], w1_ref[...],
                            preferred_element_type=jnp.float32))
    m = m * jnp.dot(rbf_ref[...], w2_ref[...],
                    preferred_element_type=jnp.float32)
    out_ref[...] = jax.nn.silu(jnp.dot(m, w3_ref[...],
                                       preferred_element_type=jnp.float32)
                               ).astype(out_ref.dtype)


def _tc2_body(gg_ref, cbf_ref, wcbf_ref, wdir_ref, wst_ref, wts_ref,
              xst_ref, xts_ref):
    gi = gg_ref[...]
    glo = jax.lax.bitcast_convert_type(gi << 16, jnp.float32)
    ghi = jax.lax.bitcast_convert_type(gi & jnp.int32(-65536), jnp.float32)
    halves = (glo[:, :64], glo[:, 64:], ghi[:, :64], ghi[:, 64:])
    cbf = cbf_ref[...]
    wcbf = wcbf_ref[...]
    acc = None
    for nb in range(4):
        p = jnp.dot(cbf[:, nb * 16:(nb + 1) * 16], wcbf,
                    preferred_element_type=jnp.float32)
        t = halves[nb] * p
        acc = t if acc is None else acc + t
    x = acc * 0.5
    x = jax.nn.silu(jnp.dot(x, wdir_ref[...],
                            preferred_element_type=jnp.float32))
    xst_ref[...] = jax.nn.silu(jnp.dot(x, wst_ref[...],
                                       preferred_element_type=jnp.float32)) * _INV_SQRT2
    xts_ref[...] = jax.nn.silu(jnp.dot(x, wts_ref[...],
                                       preferred_element_type=jnp.float32)) * _INV_SQRT2


def _full(shape):
    return pl.BlockSpec(shape, lambda i: (0, 0))


def _rows(r, d):
    return pl.BlockSpec((r, d), lambda i: (i, 0))


def kernel(m_st, rbf, cbf, idx_s, idx_swap, basis_idx1, W_m_rbf, W_rbf,
           W_m_cbf, W_cbf, W_dir, W_st, W_ts):
    E, D_EDGE = m_st.shape
    N, NB = basis_idx1.shape
    D_TRI = W_m_cbf.shape[1]
    D_RBF = rbf.shape[1]
    D_CBF = cbf.shape[2]
    R = 512
    grid = (E // R,)

    w3p = jnp.pad(W_m_cbf, ((0, 0), (0, 128 - D_TRI)))
    m_nb = pl.pallas_call(
        _tc1_body,
        grid=grid,
        in_specs=[_rows(R, D_EDGE), _rows(R, D_RBF), _full((D_EDGE, D_EDGE)),
                  _full((D_RBF, D_EDGE)), _full((D_EDGE, 128))],
        out_specs=_rows(R, 128),
        out_shape=jax.ShapeDtypeStruct((E, 128), jnp.float32),
    )(m_st, rbf, W_m_rbf, W_rbf, w3p)

    bflat = basis_idx1.reshape(-1)
    npad = (-bflat.shape[0]) % (_CH * _NW)
    bpad = jnp.concatenate([bflat, jnp.zeros((npad,), jnp.int32)])
    Gp = _sc_gather(m_nb, bpad)
    Gr = Gp[:N * NB, :D_TRI].reshape(N, NB * D_TRI)
    half = NB * D_TRI // 2
    lo = jax.lax.bitcast_convert_type(
        Gr[:, :half].astype(jnp.bfloat16), jnp.uint16).astype(jnp.uint32)
    hi = jax.lax.bitcast_convert_type(
        Gr[:, half:].astype(jnp.bfloat16), jnp.uint16).astype(jnp.uint32)
    Gpk = jax.lax.bitcast_convert_type(lo | (hi << 16), jnp.int32)

    Gg = _sc_gather(Gpk, idx_s)

    cbf2d = cbf.reshape(E, NB * D_CBF)
    xst, xts = pl.pallas_call(
        _tc2_body,
        grid=grid,
        in_specs=[_rows(R, NB * D_TRI // 2), _rows(R, NB * D_CBF),
                  _full((D_CBF, D_TRI)), _full((D_TRI, D_EDGE)),
                  _full((D_EDGE, D_EDGE)), _full((D_EDGE, D_EDGE))],
        out_specs=[_rows(R, D_EDGE), _rows(R, D_EDGE)],
        out_shape=[jax.ShapeDtypeStruct((E, D_EDGE), jnp.float32),
                   jax.ShapeDtypeStruct((E, D_EDGE), jnp.float32)],
    )(Gg, cbf2d, W_cbf, W_dir, W_st, W_ts)

    return _sc_gather_add(xst, xts, idx_swap)
